# Initial kernel scaffold; baseline (speedup 1.0000x reference)
import jax
import jax.numpy as jnp
from jax import lax
from jax.experimental import pallas as pl
from jax.experimental.pallas import tpu as pltpu
from jax.experimental.pallas import tpu_sc as plsc

N = 10000
E = 160000
D = 128
DE = 16
DU = 64
HS = 128
NB = 8

NC = 2
NS = 16
NW = NC * NS
EPW = E // NW
CH = 128
NFULL = EPW // CH
TAIL = EPW - NFULL * CH
RPT = N // NS
EXPC = 125


def _mesh():
    return plsc.VectorSubcoreMesh(core_axis_name="c", subcore_axis_name="s",
                                  num_cores=NC, num_subcores=NS)


def _gather_body(pxu, qx, src, dst, out, idxa, idxb, ra, rb, sa, sb):
    wid = lax.axis_index("c") * NS + lax.axis_index("s")
    wb = wid * EPW

    def chunk(i, carry):
        base = wb + jnp.minimum(i * CH, EPW - CH)
        pltpu.sync_copy(src.at[pl.ds(base, CH)], idxa)
        pltpu.sync_copy(dst.at[pl.ds(base, CH)], idxb)
        ca = pltpu.async_copy(pxu.at[idxa], ra, sa)
        cb = pltpu.async_copy(qx.at[idxb], rb, sb)
        ca.wait()
        cb.wait()
        pltpu.sync_copy(ra, out.at[0, pl.ds(base, CH)])
        pltpu.sync_copy(rb, out.at[1, pl.ds(base, CH)])
        return carry

    lax.fori_loop(0, NFULL + 1, chunk, None)


def _gather_call(pxu, qx, src, dst):
    fn = pl.kernel(
        _gather_body,
        out_type=jax.ShapeDtypeStruct((2, E, HS), jnp.float32),
        mesh=_mesh(),
        scratch_types=[
            pltpu.VMEM((CH,), jnp.int32),
            pltpu.VMEM((CH,), jnp.int32),
            pltpu.VMEM((CH, HS), jnp.float32),
            pltpu.VMEM((CH, HS), jnp.float32),
            pltpu.SemaphoreType.DMA,
            pltpu.SemaphoreType.DMA,
        ],
        name="edge_gather",
    )
    return fn(pxu, qx, src, dst)


def _scatter_body(eout, dst, zagg, zcnt, ones2d, pagg, pcnt,
                  idx, rows, ones_v, idx_t, rows_t, ones_t, agg_sh, cnt_sh):
    c = lax.axis_index("c")
    s = lax.axis_index("s")
    wb = (c * NS + s) * EPW

    @pl.when(s == 0)
    def _zero():
        pltpu.sync_copy(zagg, agg_sh)
        pltpu.sync_copy(zcnt, cnt_sh)

    pltpu.sync_copy(ones2d, ones_v)
    pltpu.sync_copy(ones2d.at[pl.ds(0, TAIL)], ones_t)
    plsc.subcore_barrier()

    def chunk(i, carry):
        base = wb + i * CH
        pltpu.sync_copy(dst.at[pl.ds(base, CH)], idx)
        pltpu.sync_copy(eout.at[pl.ds(base, CH)], rows)
        pltpu.sync_copy(rows, agg_sh.at[idx], add=True)
        pltpu.sync_copy(ones_v, cnt_sh.at[idx], add=True)
        return carry

    lax.fori_loop(0, NFULL, chunk, None)
    tb = wb + NFULL * CH
    pltpu.sync_copy(dst.at[pl.ds(tb, TAIL)], idx_t)
    pltpu.sync_copy(eout.at[pl.ds(tb, TAIL)], rows_t)
    pltpu.sync_copy(rows_t, agg_sh.at[idx_t], add=True)
    pltpu.sync_copy(ones_t, cnt_sh.at[idx_t], add=True)

    plsc.subcore_barrier()
    r0 = s * RPT

    def exp(i, carry):
        rb2 = r0 + i * EXPC
        pltpu.sync_copy(agg_sh.at[pl.ds(rb2, EXPC), :], rows.at[pl.ds(0, EXPC)])
        pltpu.sync_copy(rows.at[pl.ds(0, EXPC)], pagg.at[c, pl.ds(rb2, EXPC)])
        pltpu.sync_copy(cnt_sh.at[pl.ds(rb2, EXPC), :], ones_v.at[pl.ds(0, EXPC)])
        pltpu.sync_copy(ones_v.at[pl.ds(0, EXPC)], pcnt.at[c, pl.ds(rb2, EXPC)])
        return carry

    lax.fori_loop(0, RPT // EXPC, exp, None)


def _scatter_call(eout, dst, zagg, zcnt, ones2d):
    fn = pl.kernel(
        _scatter_body,
        out_type=(jax.ShapeDtypeStruct((NC, N, HS), jnp.float32),
                  jax.ShapeDtypeStruct((NC, N, DE), jnp.float32)),
        mesh=_mesh(),
        scratch_types=[
            pltpu.VMEM((CH,), jnp.int32),
            pltpu.VMEM((CH, HS), jnp.float32),
            pltpu.VMEM((CH, DE), jnp.float32),
            pltpu.VMEM((TAIL,), jnp.int32),
            pltpu.VMEM((TAIL, HS), jnp.float32),
            pltpu.VMEM((TAIL, DE), jnp.float32),
            pltpu.VMEM_SHARED((N, HS), jnp.float32),
            pltpu.VMEM_SHARED((N, DE), jnp.float32),
        ],
        name="edge_scatter",
    )
    return fn(eout, dst, zagg, zcnt, ones2d)


def _prep_kernel(x_ref, u_ref, b_ref, wsrc, wdst, wu, be1, ax, au, bn1,
                 pxu_o, qx_o, pnb_o):
    x = x_ref[...]
    u = u_ref[...]
    onehot = (b_ref[...] == lax.broadcasted_iota(jnp.int32, (N, NB), 1)
              ).astype(jnp.float32)
    dn = (((1,), (1,)), ((), ()))
    dn0 = (((1,), (0,)), ((), ()))
    pu = lax.dot_general(u, wu[...], dn, preferred_element_type=jnp.float32) + be1[...]
    pu2 = lax.dot_general(u, au[...], dn, preferred_element_type=jnp.float32) + bn1[...]
    pxu_o[...] = (lax.dot_general(x, wsrc[...], dn, preferred_element_type=jnp.float32)
                  + lax.dot_general(onehot, pu, dn0, preferred_element_type=jnp.float32))
    qx_o[...] = lax.dot_general(x, wdst[...], dn, preferred_element_type=jnp.float32)
    pnb_o[...] = (lax.dot_general(x, ax[...], dn, preferred_element_type=jnp.float32)
                  + lax.dot_general(onehot, pu2, dn0, preferred_element_type=jnp.float32))


EBLK = 2000


def _edge_kernel(g_ref, ea_ref, wea, we2, be2, out_ref):
    dn = (((1,), (1,)), ((), ()))
    pre = (g_ref[0] + g_ref[1]
           + lax.dot_general(ea_ref[...], wea[...], dn,
                             preferred_element_type=jnp.float32))
    h = jnp.maximum(pre, 0.0)
    out_ref[...] = lax.dot_general(h, we2[...], dn,
                                   preferred_element_type=jnp.float32) + be2[...]


def _edge_call(g2, ea, wea, we2, be2):
    grid = E // EBLK
    return pl.pallas_call(
        _edge_kernel,
        grid=(grid,),
        in_specs=[
            pl.BlockSpec((2, EBLK, HS), lambda i: (0, i, 0)),
            pl.BlockSpec((EBLK, DE), lambda i: (i, 0)),
            pl.BlockSpec((HS, DE), lambda i: (0, 0)),
            pl.BlockSpec((HS, HS), lambda i: (0, 0)),
            pl.BlockSpec((1, HS), lambda i: (0, 0)),
        ],
        out_specs=pl.BlockSpec((EBLK, HS), lambda i: (i, 0)),
        out_shape=jax.ShapeDtypeStruct((E, HS), jnp.float32),
        compiler_params=pltpu.CompilerParams(
            dimension_semantics=("arbitrary",)),
        name="edge_mlp",
    )(g2, ea, wea, we2, be2)


def _node_kernel(pnb, pa1, pc1, pa2, pc2, a1, a2, wn2, bn2, b_ref, u_ref,
                 bgu, bgmax, bgmean, bg1, wg2, bg2, x2_o, u2_o):
    dn = (((1,), (1,)), ((), ()))
    dnc = (((0,), (0,)), ((), ()))
    cnt1 = jnp.maximum(pc1[0, :, 0:1] + pc1[1, :, 0:1], 1.0)
    cnt2 = jnp.maximum(pc2[0, :, 0:1] + pc2[1, :, 0:1], 1.0)
    agg1 = (pa1[0] + pa1[1]) / cnt1
    agg2 = (pa2[0] + pa2[1]) / cnt2
    pre = (pnb[...]
           + lax.dot_general(agg1, a1[...], dn, preferred_element_type=jnp.float32)
           + lax.dot_general(agg2, a2[...], dn, preferred_element_type=jnp.float32))
    x2 = lax.dot_general(jnp.maximum(pre, 0.0), wn2[...], dn,
                         preferred_element_type=jnp.float32) + bn2[...]
    x2_o[...] = x2

    b2d = b_ref[...]
    onehot = (b2d == lax.broadcasted_iota(jnp.int32, (N, NB), 1)
              ).astype(jnp.float32)
    ssum = lax.dot_general(onehot, x2, dnc, preferred_element_type=jnp.float32)
    cntb = jnp.maximum(
        lax.dot_general(onehot, jnp.ones((N, 1), jnp.float32), dnc,
                        preferred_element_type=jnp.float32), 1.0)
    gmean = ssum / cntb
    parts = []
    for b in range(NB):
        m = b2d == b
        parts.append(jnp.max(jnp.where(m, x2, -jnp.inf), axis=0, keepdims=True))
    gmax = jnp.concatenate(parts, axis=0)
    preg = (lax.dot_general(u_ref[...], bgu[...], dn, preferred_element_type=jnp.float32)
            + lax.dot_general(gmax, bgmax[...], dn, preferred_element_type=jnp.float32)
            + lax.dot_general(gmean, bgmean[...], dn, preferred_element_type=jnp.float32)
            + bg1[...])
    u2_o[...] = lax.dot_general(jnp.maximum(preg, 0.0), wg2[...], dn,
                                preferred_element_type=jnp.float32) + bg2[...]


def kernel(x, edge_index1, edge_index2, edge_attr1, edge_attr2, u, batch,
           W_e1, b_e1, W_e2, b_e2, W_n1, b_n1, W_n2, b_n2, W_g1, b_g1,
           W_g2, b_g2):
    f32 = jnp.float32
    batch2d = batch.astype(jnp.int32).reshape(N, 1)
    w_src = W_e1[:, :D]
    w_dst = W_e1[:, D:2 * D]
    w_ea = W_e1[:, 2 * D:2 * D + DE]
    w_u = W_e1[:, 2 * D + DE:]
    a_x = W_n1[:, :D]
    a_1 = W_n1[:, D:D + HS]
    a_2 = W_n1[:, D + HS:D + 2 * HS]
    a_u = W_n1[:, D + 2 * HS:]
    bg_u = W_g1[:, :DU]
    bg_max = W_g1[:, DU:DU + HS]
    bg_mean = W_g1[:, DU + HS:]

    pxu, qx, pnb = pl.pallas_call(
        _prep_kernel,
        out_shape=(jax.ShapeDtypeStruct((N, HS), f32),
                   jax.ShapeDtypeStruct((N, HS), f32),
                   jax.ShapeDtypeStruct((N, HS), f32)),
        name="prep",
    )(x, u, batch2d, w_src, w_dst, w_u, b_e1.reshape(1, HS),
      a_x, a_u, b_n1.reshape(1, HS))

    zagg = jnp.zeros((N, HS), f32)
    zcnt = jnp.zeros((N, DE), f32)
    ones2d = jnp.ones((CH, DE), f32)

    outs = []
    aggs = []
    for ei, ea in ((edge_index1, edge_attr1), (edge_index2, edge_attr2)):
        src = ei[0].astype(jnp.int32)
        dst = ei[1].astype(jnp.int32)
        g2 = _gather_call(pxu, qx, src, dst)
        eout = _edge_call(g2, ea, w_ea, W_e2, b_e2.reshape(1, HS))
        outs.append(eout)
        aggs.append(_scatter_call(eout, dst, zagg, zcnt, ones2d))

    (pa1, pc1), (pa2, pc2) = aggs
    x2, u2 = pl.pallas_call(
        _node_kernel,
        out_shape=(jax.ShapeDtypeStruct((N, HS), f32),
                   jax.ShapeDtypeStruct((NB, HS), f32)),
        name="node_global",
    )(pnb, pa1, pc1, pa2, pc2, a_1, a_2, W_n2, b_n2.reshape(1, HS),
      batch2d, u, bg_u, bg_max, bg_mean, b_g1.reshape(1, HS), W_g2,
      b_g2.reshape(1, HS))

    return (x2, outs[0], outs[1], u2)


# speedup vs baseline: 5.4270x; 5.4270x over previous
import jax
import jax.numpy as jnp
from jax import lax
from jax.experimental import pallas as pl
from jax.experimental.pallas import tpu as pltpu
from jax.experimental.pallas import tpu_sc as plsc

N = 10000
E = 160000
D = 128
DE = 16
DU = 64
HS = 128
NB = 8

NC = 2
NS = 16
NW = NC * NS
EPW = E // NW
CH = 128
NFULL = EPW // CH
TAIL = EPW - NFULL * CH
NPAD = 10240
RPT = NPAD // NS
EXPC = 128


def _mesh():
    return plsc.VectorSubcoreMesh(core_axis_name="c", subcore_axis_name="s",
                                  num_cores=NC, num_subcores=NS)


def _gather_body(pxu, qx, src, dst, out, idxa, idxb, ra, rb, sa, sb):
    wid = lax.axis_index("c") * NS + lax.axis_index("s")
    wb = wid * EPW

    def chunk(i, carry):
        base = wb + jnp.minimum(i * CH, EPW - CH)
        pltpu.sync_copy(src.at[pl.ds(base, CH)], idxa)
        pltpu.sync_copy(dst.at[pl.ds(base, CH)], idxb)
        ca = pltpu.async_copy(pxu.at[idxa], ra, sa)
        cb = pltpu.async_copy(qx.at[idxb], rb, sb)
        ca.wait()
        cb.wait()
        pltpu.sync_copy(ra, out.at[0, pl.ds(base, CH)])
        pltpu.sync_copy(rb, out.at[1, pl.ds(base, CH)])
        return carry

    lax.fori_loop(0, NFULL + 1, chunk, None)


def _gather_call(pxu, qx, src, dst):
    fn = pl.kernel(
        _gather_body,
        out_type=jax.ShapeDtypeStruct((2, E, HS), jnp.float32),
        mesh=_mesh(),
        scratch_types=[
            pltpu.VMEM((CH,), jnp.int32),
            pltpu.VMEM((CH,), jnp.int32),
            pltpu.VMEM((CH, HS), jnp.float32),
            pltpu.VMEM((CH, HS), jnp.float32),
            pltpu.SemaphoreType.DMA,
            pltpu.SemaphoreType.DMA,
        ],
        name="edge_gather",
    )
    return fn(pxu, qx, src, dst)


CPW = (E // CH) // NW
XTRA = (E // CH) - CPW * NW


def _scatter_body(eout, dst, zagg, ones2d, pagg, pcnt, idx, rows, ones_r,
                  agg_sh):
    c = lax.axis_index("c")
    s = lax.axis_index("s")
    wid = c * NS + s
    base = (CPW * wid + jnp.minimum(wid, XTRA)) * CH
    r0 = s * RPT

    def zero(i, carry):
        rb = r0 + i * EXPC
        pltpu.sync_copy(zagg.at[pl.ds(rb, EXPC)], rows)
        pltpu.sync_copy(rows, agg_sh.at[pl.ds(rb, EXPC), :])
        return carry

    def export(dst_hbm):
        def exp(i, carry):
            rb = r0 + i * EXPC
            pltpu.sync_copy(agg_sh.at[pl.ds(rb, EXPC), :], rows)
            pltpu.sync_copy(rows, dst_hbm.at[c, pl.ds(rb, EXPC)])
            return carry

        lax.fori_loop(0, RPT // EXPC, exp, None)

    pltpu.sync_copy(ones2d, ones_r)
    lax.fori_loop(0, RPT // EXPC, zero, None)
    plsc.subcore_barrier()

    def chunk1(i, carry):
        eb = base + i * CH
        pltpu.sync_copy(dst.at[pl.ds(eb, CH)], idx)
        pltpu.sync_copy(eout.at[pl.ds(eb, CH)], rows)
        pltpu.sync_copy(rows, agg_sh.at[idx], add=True)
        return carry

    lax.fori_loop(0, CPW, chunk1, None)

    @pl.when(wid < XTRA)
    def _extra1():
        chunk1(CPW, None)

    plsc.subcore_barrier()
    export(pagg)
    plsc.subcore_barrier()
    lax.fori_loop(0, RPT // EXPC, zero, None)
    plsc.subcore_barrier()

    def chunk2(i, carry):
        eb = base + i * CH
        pltpu.sync_copy(dst.at[pl.ds(eb, CH)], idx)
        pltpu.sync_copy(ones_r, agg_sh.at[idx], add=True)
        return carry

    lax.fori_loop(0, CPW, chunk2, None)

    @pl.when(wid < XTRA)
    def _extra2():
        chunk2(CPW, None)

    plsc.subcore_barrier()
    export(pcnt)


def _scatter_call(eout, dst, zagg, ones2d):
    fn = pl.kernel(
        _scatter_body,
        out_type=(jax.ShapeDtypeStruct((NC, NPAD, HS), jnp.float32),
                  jax.ShapeDtypeStruct((NC, NPAD, HS), jnp.float32)),
        mesh=_mesh(),
        scratch_types=[
            pltpu.VMEM((CH,), jnp.int32),
            pltpu.VMEM((CH, HS), jnp.float32),
            pltpu.VMEM((CH, HS), jnp.float32),
            pltpu.VMEM_SHARED((NPAD, HS), jnp.float32),
        ],
        name="edge_scatter",
    )
    return fn(eout, dst, zagg, ones2d)


def _prep_kernel(x_ref, u_ref, b_ref, wsrc, wdst, wu, be1, ax, au, bn1,
                 pxu_o, qx_o, pnb_o):
    x = x_ref[...]
    u = u_ref[...]
    onehot = (b_ref[...] == lax.broadcasted_iota(jnp.int32, (N, NB), 1)
              ).astype(jnp.float32)
    dn = (((1,), (1,)), ((), ()))
    dn0 = (((1,), (0,)), ((), ()))
    pu = lax.dot_general(u, wu[...], dn, preferred_element_type=jnp.float32) + be1[...]
    pu2 = lax.dot_general(u, au[...], dn, preferred_element_type=jnp.float32) + bn1[...]
    pxu_o[...] = (lax.dot_general(x, wsrc[...], dn, preferred_element_type=jnp.float32)
                  + lax.dot_general(onehot, pu, dn0, preferred_element_type=jnp.float32))
    qx_o[...] = lax.dot_general(x, wdst[...], dn, preferred_element_type=jnp.float32)
    pnb_o[...] = (lax.dot_general(x, ax[...], dn, preferred_element_type=jnp.float32)
                  + lax.dot_general(onehot, pu2, dn0, preferred_element_type=jnp.float32))


EBLK = 2000


def _edge_kernel(g_ref, ea_ref, wea, we2, be2, out_ref):
    dn = (((1,), (1,)), ((), ()))
    pre = (g_ref[0] + g_ref[1]
           + lax.dot_general(ea_ref[...], wea[...], dn,
                             preferred_element_type=jnp.float32))
    h = jnp.maximum(pre, 0.0)
    out_ref[...] = lax.dot_general(h, we2[...], dn,
                                   preferred_element_type=jnp.float32) + be2[...]


def _edge_call(g2, ea, wea, we2, be2):
    grid = E // EBLK
    return pl.pallas_call(
        _edge_kernel,
        grid=(grid,),
        in_specs=[
            pl.BlockSpec((2, EBLK, HS), lambda i: (0, i, 0)),
            pl.BlockSpec((EBLK, DE), lambda i: (i, 0)),
            pl.BlockSpec((HS, DE), lambda i: (0, 0)),
            pl.BlockSpec((HS, HS), lambda i: (0, 0)),
            pl.BlockSpec((1, HS), lambda i: (0, 0)),
        ],
        out_specs=pl.BlockSpec((EBLK, HS), lambda i: (i, 0)),
        out_shape=jax.ShapeDtypeStruct((E, HS), jnp.float32),
        compiler_params=pltpu.CompilerParams(
            dimension_semantics=("arbitrary",)),
        name="edge_mlp",
    )(g2, ea, wea, we2, be2)


NBLK = 1280


def _node_kernel(pnb, pa1, pc1, pa2, pc2, a1, a2, wn2, bn2, x2_o):
    dn = (((1,), (1,)), ((), ()))
    cnt1 = jnp.maximum(pc1[0, :, 0:1] + pc1[1, :, 0:1], 1.0)
    cnt2 = jnp.maximum(pc2[0, :, 0:1] + pc2[1, :, 0:1], 1.0)
    agg1 = (pa1[0] + pa1[1]) / cnt1
    agg2 = (pa2[0] + pa2[1]) / cnt2
    pre = (pnb[...]
           + lax.dot_general(agg1, a1[...], dn, preferred_element_type=jnp.float32)
           + lax.dot_general(agg2, a2[...], dn, preferred_element_type=jnp.float32))
    x2_o[...] = lax.dot_general(jnp.maximum(pre, 0.0), wn2[...], dn,
                                preferred_element_type=jnp.float32) + bn2[...]


def _node_call(pnb, pa1, pc1, pa2, pc2, a1, a2, wn2, bn2):
    return pl.pallas_call(
        _node_kernel,
        grid=(NPAD // NBLK,),
        in_specs=[
            pl.BlockSpec((NBLK, HS), lambda i: (i, 0)),
            pl.BlockSpec((2, NBLK, HS), lambda i: (0, i, 0)),
            pl.BlockSpec((2, NBLK, HS), lambda i: (0, i, 0)),
            pl.BlockSpec((2, NBLK, HS), lambda i: (0, i, 0)),
            pl.BlockSpec((2, NBLK, HS), lambda i: (0, i, 0)),
            pl.BlockSpec((HS, HS), lambda i: (0, 0)),
            pl.BlockSpec((HS, HS), lambda i: (0, 0)),
            pl.BlockSpec((HS, HS), lambda i: (0, 0)),
            pl.BlockSpec((1, HS), lambda i: (0, 0)),
        ],
        out_specs=pl.BlockSpec((NBLK, HS), lambda i: (i, 0)),
        out_shape=jax.ShapeDtypeStruct((N, HS), jnp.float32),
        compiler_params=pltpu.CompilerParams(
            dimension_semantics=("arbitrary",)),
        name="node_mlp",
    )(pnb, pa1, pc1, pa2, pc2, a1, a2, wn2, bn2)


def _global_kernel(x2_ref, b_ref, u_ref, bgu, bgmax, bgmean, bg1, wg2, bg2,
                   u2_o):
    dn = (((1,), (1,)), ((), ()))
    dnc = (((0,), (0,)), ((), ()))
    x2 = x2_ref[...]
    b2d = b_ref[...]
    onehot = (b2d == lax.broadcasted_iota(jnp.int32, (N, NB), 1)
              ).astype(jnp.float32)
    ssum = lax.dot_general(onehot, x2, dnc, preferred_element_type=jnp.float32)
    cntb = jnp.maximum(
        lax.dot_general(onehot, jnp.ones((N, 1), jnp.float32), dnc,
                        preferred_element_type=jnp.float32), 1.0)
    gmean = ssum / cntb
    parts = []
    for b in range(NB):
        m = b2d == b
        parts.append(jnp.max(jnp.where(m, x2, -jnp.inf), axis=0, keepdims=True))
    gmax = jnp.concatenate(parts, axis=0)
    preg = (lax.dot_general(u_ref[...], bgu[...], dn, preferred_element_type=jnp.float32)
            + lax.dot_general(gmax, bgmax[...], dn, preferred_element_type=jnp.float32)
            + lax.dot_general(gmean, bgmean[...], dn, preferred_element_type=jnp.float32)
            + bg1[...])
    u2_o[...] = lax.dot_general(jnp.maximum(preg, 0.0), wg2[...], dn,
                                preferred_element_type=jnp.float32) + bg2[...]


def kernel(x, edge_index1, edge_index2, edge_attr1, edge_attr2, u, batch,
           W_e1, b_e1, W_e2, b_e2, W_n1, b_n1, W_n2, b_n2, W_g1, b_g1,
           W_g2, b_g2):
    f32 = jnp.float32
    batch2d = batch.astype(jnp.int32).reshape(N, 1)
    w_src = W_e1[:, :D]
    w_dst = W_e1[:, D:2 * D]
    w_ea = W_e1[:, 2 * D:2 * D + DE]
    w_u = W_e1[:, 2 * D + DE:]
    a_x = W_n1[:, :D]
    a_1 = W_n1[:, D:D + HS]
    a_2 = W_n1[:, D + HS:D + 2 * HS]
    a_u = W_n1[:, D + 2 * HS:]
    bg_u = W_g1[:, :DU]
    bg_max = W_g1[:, DU:DU + HS]
    bg_mean = W_g1[:, DU + HS:]

    pxu, qx, pnb = pl.pallas_call(
        _prep_kernel,
        out_shape=(jax.ShapeDtypeStruct((N, HS), f32),
                   jax.ShapeDtypeStruct((N, HS), f32),
                   jax.ShapeDtypeStruct((N, HS), f32)),
        name="prep",
    )(x, u, batch2d, w_src, w_dst, w_u, b_e1.reshape(1, HS),
      a_x, a_u, b_n1.reshape(1, HS))

    zagg = jnp.zeros((NPAD, HS), f32)
    ones2d = jnp.ones((CH, HS), f32)

    outs = []
    aggs = []
    for ei, ea in ((edge_index1, edge_attr1), (edge_index2, edge_attr2)):
        src = ei[0].astype(jnp.int32)
        dst = ei[1].astype(jnp.int32)
        g2 = _gather_call(pxu, qx, src, dst)
        eout = _edge_call(g2, ea, w_ea, W_e2, b_e2.reshape(1, HS))
        outs.append(eout)
        pa, pc = _scatter_call(eout, dst, zagg, ones2d)
        aggs.append((pa, pc))

    (pa1, pc1), (pa2, pc2) = aggs
    x2 = _node_call(pnb, pa1, pc1, pa2, pc2, a_1, a_2, W_n2,
                    b_n2.reshape(1, HS))
    u2 = pl.pallas_call(
        _global_kernel,
        out_shape=jax.ShapeDtypeStruct((NB, HS), f32),
        name="global_mlp",
    )(x2, batch2d, u, bg_u, bg_max, bg_mean, b_g1.reshape(1, HS), W_g2,
      b_g2.reshape(1, HS))

    return (x2, outs[0], outs[1], u2)
